# baseline (speedup 1.0000x reference)
import functools

import jax
import jax.numpy as jnp
from jax import lax
from jax.experimental import pallas as pl
from jax.experimental.pallas import tpu as pltpu
from jax.experimental.pallas import tpu_sc as plsc


def _gelu(x):
    return 0.5 * x * (1.0 + jnp.tanh(0.7978845608028654 * (x + 0.044715 * x * x * x)))


def _stats_kernel(x_ref, mr_ref, W0_ref, b0_ref, w1_ref,
                  W2_ref, b2_ref, w3_ref, cols_ref, marow_ref):
    x = x_ref[0]
    s = x.shape[0]
    w1 = w1_ref[...]
    w3 = w3_ref[...]
    mrowf = mr_ref[0].astype(jnp.float32)
    mrowb = mrowf > 0.0

    xw = jnp.dot(x.astype(jnp.bfloat16), W0_ref[...].astype(jnp.bfloat16),
                 preferred_element_type=jnp.float32)
    h = _gelu(xw + b0_ref[...])

    cdims = (((1,), (1,)), ((), ()))
    sl = jax.lax.dot_general(w1, h, cdims, preferred_element_type=jnp.float32)
    a_row = jax.lax.dot_general(w3, h, cdims,
                                preferred_element_type=jnp.float32)
    sc = _gelu(jnp.dot(h, W2_ref[...],
                       preferred_element_type=jnp.float32) + b2_ref[...])
    c_row = jax.lax.dot_general(w3, sc, cdims,
                                preferred_element_type=jnp.float32)

    slm = mrowf * sl + (mrowf - 1.0) * 10.0
    m1 = jnp.max(slm)
    z1 = jnp.sum(jnp.exp(slm - m1))
    slp = (m1 + jnp.log(z1)) - slm

    neg = jnp.float32(-1e30)
    ma = jnp.max(jnp.where(mrowb, a_row, neg))
    mc = jnp.max(jnp.where(mrowb, c_row, neg))
    m2 = jnp.maximum(ma + mc, -10.0)
    ea = jnp.where(mrowb, jnp.exp(a_row - ma), 0.0)
    ec = jnp.where(mrowb, jnp.exp(c_row - mc), 0.0)

    pad = jnp.zeros_like(mrowf)
    stack = jnp.concatenate(
        [ea, mrowf, ec, pad, pad, pad, pad, pad], axis=0)
    colsT = jnp.transpose(stack, (1, 0))
    ea_c = colsT[:, 0:1]
    ec_c = colsT[:, 2:3]

    ii = jax.lax.broadcasted_iota(jnp.int32, (s, s), 0)
    jj = jax.lax.broadcasted_iota(jnp.int32, (s, s), 1)
    tri_f = jnp.where(jj >= ii, 1.0, 0.0)
    sa_c = jax.lax.dot_general(tri_f, ea_c, (((1,), (0,)), ((), ())),
                               preferred_element_type=jnp.float32)
    z2p = jnp.sum(ec_c * sa_c)
    p = jnp.sum(mrowf)
    npairs = 0.5 * p * (p + 1.0)
    z2 = z2p * jnp.exp((ma + mc) - m2) \
        + (s * s - npairs) * jnp.exp(-10.0 - m2)
    lse2 = m2 + jnp.log(z2)

    ut_row = slp + (lse2 + 10.0)
    d_row = c_row + 10.0
    stack2 = jnp.concatenate(
        [ut_row, d_row, mrowf] + [pad] * 13, axis=0)
    cols_ref[0] = jnp.transpose(stack2, (1, 0))
    marow_ref[0] = jnp.concatenate([a_row, mrowf], axis=0)


def _make_emit(B, S):
    NC, NS = 2, 16
    NW = NC * NS
    RPW = (B * S) // NW
    NCHUNK = S // 16

    mesh = plsc.VectorSubcoreMesh(core_axis_name="c", subcore_axis_name="s")

    @functools.partial(
        pl.kernel, mesh=mesh,
        out_type=jax.ShapeDtypeStruct((B * S, S), jnp.float32),
        scratch_types=[
            pltpu.VMEM((RPW, S), jnp.float32),
            pltpu.VMEM((2, S), jnp.float32),
            pltpu.VMEM((RPW, 16), jnp.float32),
            pltpu.VMEM((S,), jnp.float32),
        ],
    )
    def emit(cols_hbm, marow_hbm, out_hbm, rows_v, ma_v, cols_v, am_v):
        wid = lax.axis_index("s") * NC + lax.axis_index("c")
        base = wid * RPW
        b = base // S
        i0 = base - b * S
        pltpu.sync_copy(cols_hbm.at[pl.ds(base, RPW)], cols_v)
        pltpu.sync_copy(marow_hbm.at[b], ma_v)
        jiota = lax.iota(jnp.int32, 16)

        for c in range(NCHUNK):
            am_v[pl.ds(c * 16, 16)] = \
                ma_v[0, pl.ds(c * 16, 16)] * ma_v[1, pl.ds(c * 16, 16)]

        def row_body(rl, _):
            ig = i0 + rl
            cv = cols_v[rl, pl.ds(0, 16)]
            ut_i = cv[0]
            mi_f = cv[2]
            dd = mi_f * cv[1]

            for c in range(NCHUNK):
                av = am_v[pl.ds(c * 16, 16)]
                mv = ma_v[1, pl.ds(c * 16, 16)]
                jv = jiota + c * 16
                full = dd * mv + mi_f * av
                rows_v[rl, pl.ds(c * 16, 16)] = \
                    ut_i - jnp.where(jv >= ig, full, 0.0)
            return 0

        lax.fori_loop(0, RPW, row_body, 0)
        pltpu.sync_copy(rows_v, out_hbm.at[pl.ds(base, RPW)])

    return emit


@jax.jit
def kernel(inputs, mask, W0, b0, w1, W2, b2, w3):
    B, S, D = inputs.shape
    U = W0.shape[1]
    mr = mask.reshape(B, 1, S)
    cols, marow = pl.pallas_call(
        _stats_kernel,
        grid=(B,),
        in_specs=[
            pl.BlockSpec((1, S, D), lambda b: (b, 0, 0)),
            pl.BlockSpec((1, 1, S), lambda b: (b, 0, 0)),
            pl.BlockSpec((D, U), lambda b: (0, 0)),
            pl.BlockSpec((1, U), lambda b: (0, 0)),
            pl.BlockSpec((1, U), lambda b: (0, 0)),
            pl.BlockSpec((U, U), lambda b: (0, 0)),
            pl.BlockSpec((1, U), lambda b: (0, 0)),
            pl.BlockSpec((1, U), lambda b: (0, 0)),
        ],
        out_specs=[
            pl.BlockSpec((1, S, 16), lambda b: (b, 0, 0)),
            pl.BlockSpec((1, 2, S), lambda b: (b, 0, 0)),
        ],
        out_shape=[
            jax.ShapeDtypeStruct((B, S, 16), jnp.float32),
            jax.ShapeDtypeStruct((B, 2, S), jnp.float32),
        ],
    )(inputs, mr, W0, b0.reshape(1, U), w1.reshape(1, U),
      W2, b2.reshape(1, U), w3.reshape(1, U))

    out = _make_emit(B, S)(cols.reshape(B * S, 16), marow)
    return out.reshape(B, S, S)


# speedup vs baseline: 3.5853x; 3.5853x over previous
import jax
import jax.numpy as jnp
from jax.experimental import pallas as pl


def _gelu(x):
    return 0.5 * x * (1.0 + jnp.tanh(0.7978845608028654 * (x + 0.044715 * x * x * x)))


def _one_batch(h, mrowf, w1, w3, W2, b2, tri_f, trib, iis, jjs):
    s = h.shape[0]
    mrowb = mrowf > 0.0

    cdims = (((1,), (1,)), ((), ()))
    sl = jax.lax.dot_general(w1, h, cdims, preferred_element_type=jnp.float32)
    a_row = jax.lax.dot_general(w3, h, cdims,
                                preferred_element_type=jnp.float32)
    sc = _gelu(jnp.dot(h, W2, preferred_element_type=jnp.float32) + b2)
    c_row = jax.lax.dot_general(w3, sc, cdims,
                                preferred_element_type=jnp.float32)

    slm = mrowf * sl + (mrowf - 1.0) * 10.0
    m1 = jnp.max(slm)
    z1 = jnp.sum(jnp.exp(slm - m1))
    slp = (m1 + jnp.log(z1)) - slm

    neg = jnp.float32(-1e30)
    ma = jnp.max(jnp.where(mrowb, a_row, neg))
    mc = jnp.max(jnp.where(mrowb, c_row, neg))
    m2 = jnp.maximum(ma + mc, -10.0)
    ea = jnp.where(mrowb, jnp.exp(a_row - ma), 0.0)
    ec = jnp.where(mrowb, jnp.exp(c_row - mc), 0.0)

    pad = jnp.zeros_like(mrowf)
    stack = jnp.concatenate(
        [ea, mrowf, ec, pad, pad, pad, pad, pad], axis=0)
    colsT = jnp.transpose(stack, (1, 0))
    ea_c = colsT[:, 0:1]
    mcolb = colsT[:, 1:2] > 0.0
    ec_c = colsT[:, 2:3]

    sa_c = jax.lax.dot_general(tri_f, ea_c, (((1,), (0,)), ((), ())),
                               preferred_element_type=jnp.float32)
    z2p = jnp.sum(ec_c * sa_c)
    p = jnp.sum(mrowf)
    npairs = 0.5 * p * (p + 1.0)
    z2 = z2p * jnp.exp((ma + mc) - m2) \
        + (s * s - npairs) * jnp.exp(-10.0 - m2)
    lse2 = m2 + jnp.log(z2)

    ut_row = slp + (lse2 + 10.0)
    d_row = c_row + 10.0
    stack2 = jnp.concatenate(
        [ut_row, d_row, pad, pad, pad, pad, pad, pad], axis=0)
    cols2 = jnp.transpose(stack2, (1, 0))
    ut_c = cols2[:, 0:1]
    d_c = cols2[:, 1:2]

    vb = trib & (mcolb & mrowb)
    return ut_c - jnp.where(vb, d_c + a_row, 0.0)


def _answer_finder_kernel(x_ref, mr_ref, W0_ref, b0_ref, w1_ref,
                          W2_ref, b2_ref, w3_ref, out_ref):
    nb = x_ref.shape[0]
    s = out_ref.shape[1]
    w1 = w1_ref[...]
    w3 = w3_ref[...]
    W2 = W2_ref[...]
    b2 = b2_ref[...]

    xall = x_ref[...].reshape(nb * s, x_ref.shape[2])
    hall = _gelu(jnp.dot(xall.astype(jnp.bfloat16),
                         W0_ref[...].astype(jnp.bfloat16),
                         preferred_element_type=jnp.float32) + b0_ref[...])

    ii = jax.lax.broadcasted_iota(jnp.int32, (s, s), 0)
    jj = jax.lax.broadcasted_iota(jnp.int32, (s, s), 1)
    trib = jj >= ii
    tri_f = jnp.where(trib, 1.0, 0.0)

    for bb in range(nb):
        h = hall[bb * s:(bb + 1) * s, :]
        mrowf = mr_ref[bb].astype(jnp.float32)
        out_ref[bb] = _one_batch(h, mrowf, w1, w3, W2, b2,
                                 tri_f, trib, ii, jj)


@jax.jit
def kernel(inputs, mask, W0, b0, w1, W2, b2, w3):
    B, S, D = inputs.shape
    U = W0.shape[1]
    NB = 2
    mr = mask.reshape(B, 1, S)
    return pl.pallas_call(
        _answer_finder_kernel,
        grid=(B // NB,),
        in_specs=[
            pl.BlockSpec((NB, S, D), lambda b: (b, 0, 0)),
            pl.BlockSpec((NB, 1, S), lambda b: (b, 0, 0)),
            pl.BlockSpec((D, U), lambda b: (0, 0)),
            pl.BlockSpec((1, U), lambda b: (0, 0)),
            pl.BlockSpec((1, U), lambda b: (0, 0)),
            pl.BlockSpec((U, U), lambda b: (0, 0)),
            pl.BlockSpec((1, U), lambda b: (0, 0)),
            pl.BlockSpec((1, U), lambda b: (0, 0)),
        ],
        out_specs=pl.BlockSpec((NB, S, S), lambda b: (b, 0, 0)),
        out_shape=jax.ShapeDtypeStruct((B, S, S), jnp.float32),
    )(inputs, mr, W0, b0.reshape(1, U), w1.reshape(1, U),
      W2, b2.reshape(1, U), w3.reshape(1, U))


# speedup vs baseline: 3.6771x; 1.0256x over previous
import jax
import jax.numpy as jnp
from jax.experimental import pallas as pl


def _gelu(x):
    return 0.5 * x * (1.0 + jnp.tanh(0.7978845608028654 * (x + 0.044715 * x * x * x)))


def _one_batch(h, mrowf, w1, w3, W2, b2, tri_f, trib, iis, jjs):
    s = h.shape[0]
    mrowb = mrowf > 0.0

    cdims = (((1,), (1,)), ((), ()))
    sl = jax.lax.dot_general(w1, h, cdims, preferred_element_type=jnp.float32)
    a_row = jax.lax.dot_general(w3, h, cdims,
                                preferred_element_type=jnp.float32)
    sc = _gelu(jnp.dot(h, W2, preferred_element_type=jnp.float32) + b2)
    c_row = jax.lax.dot_general(w3, sc, cdims,
                                preferred_element_type=jnp.float32)

    slm = mrowf * sl + (mrowf - 1.0) * 10.0
    m1 = jnp.max(slm)
    z1 = jnp.sum(jnp.exp(slm - m1))
    slp = (m1 + jnp.log(z1)) - slm

    neg = jnp.float32(-1e30)
    ma = jnp.max(jnp.where(mrowb, a_row, neg))
    mc = jnp.max(jnp.where(mrowb, c_row, neg))
    m2 = jnp.maximum(ma + mc, -10.0)
    ea = jnp.where(mrowb, jnp.exp(a_row - ma), 0.0)
    ec = jnp.where(mrowb, jnp.exp(c_row - mc), 0.0)

    pad = jnp.zeros_like(mrowf)
    stack = jnp.concatenate(
        [ea, mrowf, ec, slp, c_row, pad, pad, pad], axis=0)
    colsT = jnp.transpose(stack, (1, 0))
    ea_c = colsT[:, 0:1]
    mcolb = colsT[:, 1:2] > 0.0
    ec_c = colsT[:, 2:3]
    slp_c = colsT[:, 3:4]
    c_c = colsT[:, 4:5]

    sa_c = jax.lax.dot_general(tri_f, ea_c, (((1,), (0,)), ((), ())),
                               preferred_element_type=jnp.float32)
    z2p = jnp.sum(ec_c * sa_c)
    p = jnp.sum(mrowf)
    npairs = 0.5 * p * (p + 1.0)
    z2 = z2p * jnp.exp((ma + mc) - m2) \
        + (s * s - npairs) * jnp.exp(-10.0 - m2)
    lse2 = m2 + jnp.log(z2)

    ut_c = slp_c + (lse2 + 10.0)
    d_c = c_c + 10.0

    vb = trib & (mcolb & mrowb)
    return ut_c - jnp.where(vb, d_c + a_row, 0.0)


def _answer_finder_kernel(x_ref, mr_ref, W0_ref, b0_ref, w1_ref,
                          W2_ref, b2_ref, w3_ref, out_ref):
    nb = x_ref.shape[0]
    s = out_ref.shape[1]
    w1 = w1_ref[...]
    w3 = w3_ref[...]
    W2 = W2_ref[...]
    b2 = b2_ref[...]

    xall = x_ref[...].reshape(nb * s, x_ref.shape[2])
    hall = _gelu(jnp.dot(xall.astype(jnp.bfloat16),
                         W0_ref[...].astype(jnp.bfloat16),
                         preferred_element_type=jnp.float32) + b0_ref[...])

    ii = jax.lax.broadcasted_iota(jnp.int32, (s, s), 0)
    jj = jax.lax.broadcasted_iota(jnp.int32, (s, s), 1)
    trib = jj >= ii
    tri_f = jnp.where(trib, 1.0, 0.0)

    for bb in range(nb):
        h = hall[bb * s:(bb + 1) * s, :]
        mrowf = mr_ref[bb].astype(jnp.float32)
        out_ref[bb] = _one_batch(h, mrowf, w1, w3, W2, b2,
                                 tri_f, trib, ii, jj)


@jax.jit
def kernel(inputs, mask, W0, b0, w1, W2, b2, w3):
    B, S, D = inputs.shape
    U = W0.shape[1]
    NB = 2
    mr = mask.reshape(B, 1, S)
    return pl.pallas_call(
        _answer_finder_kernel,
        grid=(B // NB,),
        in_specs=[
            pl.BlockSpec((NB, S, D), lambda b: (b, 0, 0)),
            pl.BlockSpec((NB, 1, S), lambda b: (b, 0, 0)),
            pl.BlockSpec((D, U), lambda b: (0, 0)),
            pl.BlockSpec((1, U), lambda b: (0, 0)),
            pl.BlockSpec((1, U), lambda b: (0, 0)),
            pl.BlockSpec((U, U), lambda b: (0, 0)),
            pl.BlockSpec((1, U), lambda b: (0, 0)),
            pl.BlockSpec((1, U), lambda b: (0, 0)),
        ],
        out_specs=pl.BlockSpec((NB, S, S), lambda b: (b, 0, 0)),
        out_shape=jax.ShapeDtypeStruct((B, S, S), jnp.float32),
    )(inputs, mr, W0, b0.reshape(1, U), w1.reshape(1, U),
      W2, b2.reshape(1, U), w3.reshape(1, U))


# speedup vs baseline: 4.2282x; 1.1499x over previous
import jax
import jax.numpy as jnp
from jax.experimental import pallas as pl


def _gelu(x):
    return 0.5 * x * (1.0 + jnp.tanh(0.7978845608028654 * (x + 0.044715 * x * x * x)))


def _one_batch(h, mrowf, w1, w3, W2, b2, tri_f, trib, iis, jjs):
    s = h.shape[0]
    mrowb = mrowf > 0.0

    cdims = (((1,), (1,)), ((), ()))
    sl = jax.lax.dot_general(w1, h, cdims, preferred_element_type=jnp.float32)
    a_row = jax.lax.dot_general(w3, h, cdims,
                                preferred_element_type=jnp.float32)
    sc = _gelu(jnp.dot(h, W2, preferred_element_type=jnp.float32) + b2)
    c_row = jax.lax.dot_general(w3, sc, cdims,
                                preferred_element_type=jnp.float32)

    slm = mrowf * sl + (mrowf - 1.0) * 10.0
    m1 = jnp.max(slm)
    z1 = jnp.sum(jnp.exp(slm - m1))

    neg = jnp.float32(-1e30)
    ma = jnp.max(jnp.where(mrowb, a_row, neg))
    mc = jnp.max(jnp.where(mrowb, c_row, neg))
    m2 = jnp.maximum(ma + mc, -10.0)
    ea = jnp.where(mrowb, jnp.exp(a_row - ma), 0.0)
    ec = jnp.where(mrowb, jnp.exp(c_row - mc), 0.0)

    pad = jnp.zeros_like(mrowf)
    stack = jnp.concatenate(
        [mrowf, slm, c_row, pad, pad, pad, pad, pad], axis=0)
    colsT = jnp.transpose(stack, (1, 0))
    mcolb = colsT[:, 0:1] > 0.0
    slm_c = colsT[:, 1:2]
    c_c = colsT[:, 2:3]

    sa_row = jax.lax.dot_general(ea, tri_f, (((1,), (1,)), ((), ())),
                                 preferred_element_type=jnp.float32)
    z2p = jnp.sum(ec * sa_row)
    p = jnp.sum(mrowf)
    npairs = 0.5 * p * (p + 1.0)
    z2 = z2p * jnp.exp((ma + mc) - m2) \
        + (s * s - npairs) * jnp.exp(-10.0 - m2)
    lse2 = m2 + jnp.log(z2)

    ut_c = (m1 + jnp.log(z1) + lse2 + 10.0) - slm_c
    d_c = c_c + 10.0

    vb = trib & (mcolb & mrowb)
    return ut_c - jnp.where(vb, d_c + a_row, 0.0)


def _answer_finder_kernel(x_ref, mr_ref, W0_ref, b0_ref, w1_ref,
                          W2_ref, b2_ref, w3_ref, out_ref):
    nb = x_ref.shape[0]
    s = out_ref.shape[1]
    w1 = w1_ref[...]
    w3 = w3_ref[...]
    W2 = W2_ref[...]
    b2 = b2_ref[...]

    xall = x_ref[...].reshape(nb * s, x_ref.shape[2])
    hall = _gelu(jnp.dot(xall.astype(jnp.bfloat16),
                         W0_ref[...].astype(jnp.bfloat16),
                         preferred_element_type=jnp.float32) + b0_ref[...])

    ii = jax.lax.broadcasted_iota(jnp.int32, (s, s), 0)
    jj = jax.lax.broadcasted_iota(jnp.int32, (s, s), 1)
    trib = jj >= ii
    tri_f = jnp.where(trib, 1.0, 0.0)

    for bb in range(nb):
        h = hall[bb * s:(bb + 1) * s, :]
        mrowf = mr_ref[bb].astype(jnp.float32)
        out_ref[bb] = _one_batch(h, mrowf, w1, w3, W2, b2,
                                 tri_f, trib, ii, jj)


@jax.jit
def kernel(inputs, mask, W0, b0, w1, W2, b2, w3):
    B, S, D = inputs.shape
    U = W0.shape[1]
    NB = 2
    mr = mask.reshape(B, 1, S)
    return pl.pallas_call(
        _answer_finder_kernel,
        grid=(B // NB,),
        in_specs=[
            pl.BlockSpec((NB, S, D), lambda b: (b, 0, 0)),
            pl.BlockSpec((NB, 1, S), lambda b: (b, 0, 0)),
            pl.BlockSpec((D, U), lambda b: (0, 0)),
            pl.BlockSpec((1, U), lambda b: (0, 0)),
            pl.BlockSpec((1, U), lambda b: (0, 0)),
            pl.BlockSpec((U, U), lambda b: (0, 0)),
            pl.BlockSpec((1, U), lambda b: (0, 0)),
            pl.BlockSpec((1, U), lambda b: (0, 0)),
        ],
        out_specs=pl.BlockSpec((NB, S, S), lambda b: (b, 0, 0)),
        out_shape=jax.ShapeDtypeStruct((B, S, S), jnp.float32),
    )(inputs, mr, W0, b0.reshape(1, U), w1.reshape(1, U),
      W2, b2.reshape(1, U), w3.reshape(1, U))
